# baseline (speedup 1.0000x reference)
import functools

import jax
import jax.numpy as jnp
from jax import lax
from jax.experimental import pallas as pl
from jax.experimental.pallas import tpu as pltpu
from jax.experimental.pallas import tpu_sc as plsc

BATCH = 4096
SEQ = 50
EMBED = 64
HIDDEN = 128
LABELS = 32

NC = 2
NS = 16
NW = NC * NS
ROWS_PER_W = BATCH // NW
CHUNKS = ROWS_PER_W
NBUF = 8


def _sc_pool_body(ids_hbm, table_hbm, out_hbm, idx_v, rows_v, acc_v, *sems):
    c = lax.axis_index("c")
    s = lax.axis_index("s")
    wid = s * NC + c
    pltpu.sync_copy(ids_hbm.at[pl.ds(wid * ROWS_PER_W, ROWS_PER_W)], idx_v)

    def start(chunk, b):
        pltpu.async_copy(table_hbm.at[idx_v.at[chunk]], rows_v.at[b], sems[b])

    def wait(b):
        pltpu.make_async_copy(
            table_hbm.at[idx_v.at[0]], rows_v.at[b], sems[b]
        ).wait()

    for b in range(NBUF):
        start(b, b)

    def group(g, carry):
        for b in range(NBUF):
            cur = g * NBUF + b
            wait(b)
            for cc in range(EMBED // 16):
                sl = pl.ds(cc * 16, 16)
                a = [rows_v[b, t, sl] for t in range(4)]
                for t in range(4, SEQ - 2, 4):
                    for j in range(4):
                        a[j] = a[j] + rows_v[b, t + j, sl]
                a[0] = a[0] + rows_v[b, SEQ - 2, sl]
                a[1] = a[1] + rows_v[b, SEQ - 1, sl]
                acc_v[cur, sl] = (a[0] + a[1]) + (a[2] + a[3])
            nxt = cur + NBUF

            @pl.when(nxt < CHUNKS)
            def _():
                start(nxt, b)

        return carry

    lax.fori_loop(0, CHUNKS // NBUF, group, 0)
    pltpu.sync_copy(acc_v, out_hbm.at[pl.ds(wid * ROWS_PER_W, ROWS_PER_W)])


_sc_pool = functools.partial(
    pl.kernel,
    out_type=jax.ShapeDtypeStruct((BATCH, EMBED), jnp.float32),
    mesh=plsc.VectorSubcoreMesh(core_axis_name="c", subcore_axis_name="s"),
    scratch_types=[
        pltpu.VMEM((ROWS_PER_W, SEQ), jnp.int32),
        pltpu.VMEM((NBUF, SEQ, EMBED), jnp.float32),
        pltpu.VMEM((ROWS_PER_W, EMBED), jnp.float32),
    ] + [pltpu.SemaphoreType.DMA] * NBUF,
    compiler_params=pltpu.CompilerParams(use_tc_tiling_on_sc=False),
)(_sc_pool_body)


VOCAB = 100000
VB = 1024


def _detile_body(tabT_ref, out_ref):
    r = lax.broadcasted_iota(jnp.int32, (EMBED, EMBED), 0)
    c = lax.broadcasted_iota(jnp.int32, (EMBED, EMBED), 1)
    ident = (r == c).astype(jnp.float32)
    t = lax.dot_general(tabT_ref[...], ident, (((0,), (0,)), ((), ())),
                        preferred_element_type=jnp.float32)
    lo = lax.slice(t, (0, 0), (VB // 2, EMBED))
    hi = lax.slice(t, (VB // 2, 0), (VB, EMBED))
    out_ref[...] = jnp.concatenate([lo, hi], axis=1)


NBLK_DT = (VOCAB + VB - 1) // VB
VOCAB_PAD = NBLK_DT * VB


def _detile(tabT):
    return pl.pallas_call(
        _detile_body,
        grid=(NBLK_DT,),
        in_specs=[pl.BlockSpec((EMBED, VB), lambda i: (0, i))],
        out_specs=pl.BlockSpec((VB // 2, 2 * EMBED), lambda i: (i, 0)),
        out_shape=jax.ShapeDtypeStruct((VOCAB_PAD // 2, 2 * EMBED),
                                       jnp.float32),
    )(tabT)


def _perm(v):
    r = jnp.bitwise_and(v, VB - 1)
    return (v - r) + jnp.bitwise_and(2 * r, VB - 1) + (r >> 9)


def _tc_head_body(ids_ref, psum_ref, gamma_ref, beta_ref, w1_ref, b1_ref,
                  w2_ref, b2_ref, out_ref):
    ids = ids_ref[...]
    cnt = jnp.sum((ids != 0).astype(jnp.float32), axis=1, keepdims=True)
    pooled = psum_ref[...] / jnp.maximum(cnt, 1.0)
    mean = jnp.mean(pooled, axis=1, keepdims=True)
    centered = pooled - mean
    var = jnp.mean(centered * centered, axis=1, keepdims=True)
    normed = centered * lax.rsqrt(var + 1e-5) * gamma_ref[...] + beta_ref[...]
    h = jnp.dot(normed, w1_ref[...], preferred_element_type=jnp.float32)
    h = jnp.maximum(h + b1_ref[...], 0.0)
    out = jnp.dot(h, w2_ref[...], preferred_element_type=jnp.float32)
    out_ref[...] = out + b2_ref[...]


def _tc_head(token_ids, psum, gamma, beta, W1, b1, W2, b2):
    blk = 512
    grid = BATCH // blk
    return pl.pallas_call(
        _tc_head_body,
        grid=(grid,),
        in_specs=[
            pl.BlockSpec((blk, SEQ), lambda i: (i, 0)),
            pl.BlockSpec((blk, EMBED), lambda i: (i, 0)),
            pl.BlockSpec((1, EMBED), lambda i: (0, 0)),
            pl.BlockSpec((1, EMBED), lambda i: (0, 0)),
            pl.BlockSpec((EMBED, HIDDEN), lambda i: (0, 0)),
            pl.BlockSpec((1, HIDDEN), lambda i: (0, 0)),
            pl.BlockSpec((HIDDEN, LABELS), lambda i: (0, 0)),
            pl.BlockSpec((1, LABELS), lambda i: (0, 0)),
        ],
        out_specs=pl.BlockSpec((blk, LABELS), lambda i: (i, 0)),
        out_shape=jax.ShapeDtypeStruct((BATCH, LABELS), jnp.float32),
    )(token_ids, psum, gamma, beta, W1, b1, W2, b2)


def kernel(token_ids, table, gamma, beta, W1, b1, W2, b2):
    ids32 = token_ids.astype(jnp.int32)
    tab_lin = _detile(table.T).reshape(VOCAB_PAD, EMBED)
    psum = _sc_pool(_perm(ids32), tab_lin)
    return _tc_head(ids32, psum,
                    gamma.reshape(1, EMBED), beta.reshape(1, EMBED),
                    W1, b1.reshape(1, HIDDEN), W2, b2.reshape(1, LABELS))
